# baseline (speedup 1.0000x reference)
import functools

import jax
import jax.numpy as jnp
from jax import lax
from jax.experimental import pallas as pl
from jax.experimental.pallas import tpu as pltpu
from jax.experimental.pallas import tpu_sc as plsc

NC = 2
NS = 16
L = 16
NW = NC * NS
B = 80


def _mm_bias(x, W, b):
    M, K = x.shape
    Nf = W.shape[1]
    BM = 2000

    def body(x_ref, W_ref, b_ref, o_ref):
        o_ref[...] = (
            jnp.dot(x_ref[...], W_ref[...], preferred_element_type=jnp.float32)
            + b_ref[...]
        )

    return pl.pallas_call(
        body,
        grid=(M // BM,),
        in_specs=[
            pl.BlockSpec((BM, K), lambda i: (i, 0)),
            pl.BlockSpec((K, Nf), lambda i: (0, 0)),
            pl.BlockSpec((1, Nf), lambda i: (0, 0)),
        ],
        out_specs=pl.BlockSpec((BM, Nf), lambda i: (i, 0)),
        out_shape=jax.ShapeDtypeStruct((M, Nf), jnp.float32),
    )(x, W, b.reshape(1, Nf))


def _combine_prelu_mm(p0, p1, a, W, b):
    M, K = p0.shape
    Nf = W.shape[1]
    BM = 2000

    def body(p0_ref, p1_ref, a_ref, W_ref, b_ref, o_ref):
        h = p0_ref[...] + p1_ref[...]
        h = jnp.where(h >= 0, h, a_ref[...] * h)
        o_ref[...] = (
            jnp.dot(h, W_ref[...], preferred_element_type=jnp.float32) + b_ref[...]
        )

    return pl.pallas_call(
        body,
        grid=(M // BM,),
        in_specs=[
            pl.BlockSpec((BM, K), lambda i: (i, 0)),
            pl.BlockSpec((BM, K), lambda i: (i, 0)),
            pl.BlockSpec((1, K), lambda i: (0, 0)),
            pl.BlockSpec((K, Nf), lambda i: (0, 0)),
            pl.BlockSpec((1, Nf), lambda i: (0, 0)),
        ],
        out_specs=pl.BlockSpec((BM, Nf), lambda i: (i, 0)),
        out_shape=jax.ShapeDtypeStruct((M, Nf), jnp.float32),
    )(p0, p1, a.reshape(1, K), W, b.reshape(1, Nf))


def _combine_prelu(p0, p1, a):
    M, K = p0.shape
    BM = 2000

    def body(p0_ref, p1_ref, a_ref, o_ref):
        h = p0_ref[...] + p1_ref[...]
        o_ref[...] = jnp.where(h >= 0, h, a_ref[...] * h)

    return pl.pallas_call(
        body,
        grid=(M // BM,),
        in_specs=[
            pl.BlockSpec((BM, K), lambda i: (i, 0)),
            pl.BlockSpec((BM, K), lambda i: (i, 0)),
            pl.BlockSpec((1, K), lambda i: (0, 0)),
        ],
        out_specs=pl.BlockSpec((BM, K), lambda i: (i, 0)),
        out_shape=jax.ShapeDtypeStruct((M, K), jnp.float32),
    )(p0, p1, a.reshape(1, K))


def _sc_spmm(support, src3, dst3, w3):
    N, F = support.shape
    NB = dst3.shape[1]
    WB = 32
    SW = NB // WB
    RPS = (N // NS) // 8 * 8
    TAIL = N - NS * RPS
    ZR = 48
    mesh = plsc.VectorSubcoreMesh(core_axis_name="c", subcore_axis_name="s")

    @functools.partial(
        pl.kernel,
        out_type=jax.ShapeDtypeStruct((NC, N, F), jnp.float32),
        mesh=mesh,
        scratch_types=[
            pltpu.VMEM_SHARED((N, F), jnp.float32),
            pltpu.VMEM((NB * B,), jnp.int32),
            pltpu.VMEM((WB, B), jnp.int32),
            pltpu.VMEM((NB * B,), jnp.float32),
            pltpu.VMEM((B, F), jnp.float32),
            pltpu.VMEM((B, F), jnp.float32),
            pltpu.SemaphoreType.DMA,
            pltpu.SemaphoreType.DMA,
        ],
    )
    def k(sup_hbm, src_hbm, dst_hbm, w_hbm, out_hbm,
          acc, src_v, dw0, w_v, rows0, rows1, gs0, gs1):
        cid = lax.axis_index("c")
        sid = lax.axis_index("s")
        wid = cid * NS + sid

        pltpu.sync_copy(src_hbm.at[wid], src_v)
        pltpu.sync_copy(w_hbm.at[wid], w_v)

        zero = jnp.zeros((L,), jnp.float32)

        def zb(r, carry):
            for kk in range(F // L):
                rows0[r, pl.ds(kk * L, L)] = zero
            return carry

        lax.fori_loop(0, ZR, zb, 0)
        for i in range(RPS // ZR):
            pltpu.sync_copy(
                rows0.at[pl.ds(0, ZR)], acc.at[pl.ds(sid * RPS + i * ZR, ZR)]
            )

        @pl.when(sid == NS - 1)
        def _zero_tail():
            pltpu.sync_copy(rows0.at[pl.ds(0, TAIL)], acc.at[pl.ds(NS * RPS, TAIL)])

        plsc.subcore_barrier()

        def gather(j, rows, sem):
            pltpu.async_copy(sup_hbm.at[src_v.at[pl.ds(j * B, B)]], rows, sem)

        def gwait(rows, sem):
            pltpu.make_async_copy(
                sup_hbm.at[src_v.at[pl.ds(0, B)]], rows, sem
            ).wait()

        def scale(rows, j):
            def group_body(g, c2):
                wchunk = w_v[pl.ds(j * B + g * L, L)]
                for i in range(L):
                    wb = jnp.broadcast_to(wchunk[i], (L,))
                    b = g * L + i
                    for kk in range(F // L):
                        sl = pl.ds(kk * L, L)
                        rows[b, sl] = rows[b, sl] * wb
                return c2

            lax.fori_loop(0, B // L, group_body, 0)

        def win_body(w, carry):
            woff = pl.multiple_of(w * WB, WB)
            pltpu.sync_copy(dst_hbm.at[wid, pl.ds(woff, WB)], dw0)
            base = w * WB

            def pair_body(p, carry2):
                j0 = base + 2 * p
                gather(j0, rows0, gs0)
                gwait(rows0, gs0)
                scale(rows0, j0)
                pltpu.sync_copy(rows0, acc.at[dw0.at[2 * p]], add=True)

                gather(j0 + 1, rows1, gs1)
                gwait(rows1, gs1)
                scale(rows1, j0 + 1)
                pltpu.sync_copy(rows1, acc.at[dw0.at[2 * p + 1]], add=True)
                return carry2

            lax.fori_loop(0, WB // 2, pair_body, 0)
            return carry

        lax.fori_loop(0, SW, win_body, 0)
        plsc.subcore_barrier()

        pltpu.sync_copy(
            acc.at[pl.ds(sid * RPS, RPS)],
            out_hbm.at[cid, pl.ds(sid * RPS, RPS)],
        )

        @pl.when(sid == NS - 1)
        def _drain_tail():
            pltpu.sync_copy(
                acc.at[pl.ds(NS * RPS, TAIL)],
                out_hbm.at[cid, pl.ds(NS * RPS, TAIL)],
            )

    return k(support, src3, dst3, w3)


def kernel(x, edge_index, edge_weight, W1, b1, a1, W2, b2, a2):
    N, F = x.shape
    E = edge_weight.shape[0]
    NB = -(-E // (NW * B * 32)) * 32
    pad = NW * NB * B - E
    src = jnp.concatenate(
        [edge_index[0].astype(jnp.int32), jnp.zeros((pad,), jnp.int32)]
    )
    dst = jnp.concatenate(
        [edge_index[1].astype(jnp.int32), jnp.zeros((pad,), jnp.int32)]
    )
    w = jnp.concatenate([edge_weight, jnp.zeros((pad,), jnp.float32)])
    src3 = src.reshape(NW, NB * B)
    dst3 = dst.reshape(NW, NB, B)
    w3 = w.reshape(NW, NB * B)

    sup1 = _mm_bias(x, W1, b1)
    parts1 = _sc_spmm(sup1, src3, dst3, w3)
    sup2 = _combine_prelu_mm(parts1[0], parts1[1], a1, W2, b2)
    parts2 = _sc_spmm(sup2, src3, dst3, w3)
    return _combine_prelu(parts2[0], parts2[1], a2)


# speedup vs baseline: 1.0463x; 1.0463x over previous
import functools

import jax
import jax.numpy as jnp
from jax import lax
from jax.experimental import pallas as pl
from jax.experimental.pallas import tpu as pltpu
from jax.experimental.pallas import tpu_sc as plsc

NC = 2
NS = 16
L = 16
NW = NC * NS
B = 80


def _mm_bias(x, W, b):
    M, K = x.shape
    Nf = W.shape[1]
    BM = 2000

    def body(x_ref, W_ref, b_ref, o_ref):
        o_ref[...] = (
            jnp.dot(x_ref[...], W_ref[...], preferred_element_type=jnp.float32)
            + b_ref[...]
        )

    return pl.pallas_call(
        body,
        grid=(M // BM,),
        in_specs=[
            pl.BlockSpec((BM, K), lambda i: (i, 0)),
            pl.BlockSpec((K, Nf), lambda i: (0, 0)),
            pl.BlockSpec((1, Nf), lambda i: (0, 0)),
        ],
        out_specs=pl.BlockSpec((BM, Nf), lambda i: (i, 0)),
        out_shape=jax.ShapeDtypeStruct((M, Nf), jnp.float32),
    )(x, W, b.reshape(1, Nf))


def _combine_prelu_mm(p0, p1, a, W, b):
    M, K = p0.shape
    Nf = W.shape[1]
    BM = 2000

    def body(p0_ref, p1_ref, a_ref, W_ref, b_ref, o_ref):
        h = p0_ref[...] + p1_ref[...]
        h = jnp.where(h >= 0, h, a_ref[...] * h)
        o_ref[...] = (
            jnp.dot(h, W_ref[...], preferred_element_type=jnp.float32) + b_ref[...]
        )

    return pl.pallas_call(
        body,
        grid=(M // BM,),
        in_specs=[
            pl.BlockSpec((BM, K), lambda i: (i, 0)),
            pl.BlockSpec((BM, K), lambda i: (i, 0)),
            pl.BlockSpec((1, K), lambda i: (0, 0)),
            pl.BlockSpec((K, Nf), lambda i: (0, 0)),
            pl.BlockSpec((1, Nf), lambda i: (0, 0)),
        ],
        out_specs=pl.BlockSpec((BM, Nf), lambda i: (i, 0)),
        out_shape=jax.ShapeDtypeStruct((M, Nf), jnp.float32),
    )(p0, p1, a.reshape(1, K), W, b.reshape(1, Nf))


def _combine_prelu(p0, p1, a):
    M, K = p0.shape
    BM = 2000

    def body(p0_ref, p1_ref, a_ref, o_ref):
        h = p0_ref[...] + p1_ref[...]
        o_ref[...] = jnp.where(h >= 0, h, a_ref[...] * h)

    return pl.pallas_call(
        body,
        grid=(M // BM,),
        in_specs=[
            pl.BlockSpec((BM, K), lambda i: (i, 0)),
            pl.BlockSpec((BM, K), lambda i: (i, 0)),
            pl.BlockSpec((1, K), lambda i: (0, 0)),
        ],
        out_specs=pl.BlockSpec((BM, K), lambda i: (i, 0)),
        out_shape=jax.ShapeDtypeStruct((M, K), jnp.float32),
    )(p0, p1, a.reshape(1, K))


def _sc_spmm(support, src3, dst3, w3):
    N, F = support.shape
    NB = dst3.shape[1]
    WB = 32
    SW = NB // WB
    RPS = (N // NS) // 8 * 8
    TAIL = N - NS * RPS
    ZR = 48
    mesh = plsc.VectorSubcoreMesh(core_axis_name="c", subcore_axis_name="s")

    @functools.partial(
        pl.kernel,
        out_type=jax.ShapeDtypeStruct((NC, N, F), jnp.float32),
        mesh=mesh,
        scratch_types=[
            pltpu.VMEM_SHARED((N, F), jnp.float32),
            pltpu.VMEM((NB * B,), jnp.int32),
            pltpu.VMEM((WB, B), jnp.int32),
            pltpu.VMEM((NB * B,), jnp.float32),
            pltpu.VMEM((B, F), jnp.float32),
            pltpu.VMEM((B, F), jnp.float32),
            pltpu.SemaphoreType.DMA,
            pltpu.SemaphoreType.DMA,
        ],
    )
    def k(sup_hbm, src_hbm, dst_hbm, w_hbm, out_hbm,
          acc, src_v, dw0, w_v, rows0, rows1, gs0, gs1):
        cid = lax.axis_index("c")
        sid = lax.axis_index("s")
        wid = cid * NS + sid

        pltpu.sync_copy(src_hbm.at[wid], src_v)
        pltpu.sync_copy(w_hbm.at[wid], w_v)

        zero = jnp.zeros((L,), jnp.float32)

        def zb(r, carry):
            for kk in range(F // L):
                rows0[r, pl.ds(kk * L, L)] = zero
            return carry

        lax.fori_loop(0, ZR, zb, 0)
        for i in range(RPS // ZR):
            pltpu.sync_copy(
                rows0.at[pl.ds(0, ZR)], acc.at[pl.ds(sid * RPS + i * ZR, ZR)]
            )

        @pl.when(sid == NS - 1)
        def _zero_tail():
            pltpu.sync_copy(rows0.at[pl.ds(0, TAIL)], acc.at[pl.ds(NS * RPS, TAIL)])

        plsc.subcore_barrier()

        def gather(j, rows, sem):
            return pltpu.async_copy(sup_hbm.at[src_v.at[pl.ds(j * B, B)]], rows, sem)

        def gwait(rows, sem):
            pltpu.make_async_copy(
                sup_hbm.at[src_v.at[pl.ds(0, B)]], rows, sem
            ).wait()

        def scale(rows, j):
            def group_body(g, c2):
                wchunk = w_v[pl.ds(j * B + g * L, L)]
                for i in range(L):
                    wb = jnp.broadcast_to(wchunk[i], (L,))
                    b = g * L + i
                    for kk in range(F // L):
                        sl = pl.ds(kk * L, L)
                        rows[b, sl] = rows[b, sl] * wb
                return c2

            lax.fori_loop(0, B // L, group_body, 0)

        def win_body(w, carry):
            woff = pl.multiple_of(w * WB, WB)
            pltpu.sync_copy(dst_hbm.at[wid, pl.ds(woff, WB)], dw0)
            base = w * WB

            def pair_body(p, carry2):
                j0 = base + 2 * p
                d0 = gather(j0, rows0, gs0)
                d1 = gather(j0 + 1, rows1, gs1)
                d0.wait()
                scale(rows0, j0)
                pltpu.sync_copy(rows0, acc.at[dw0.at[2 * p]], add=True)

                d1.wait()
                scale(rows1, j0 + 1)
                pltpu.sync_copy(rows1, acc.at[dw0.at[2 * p + 1]], add=True)
                return carry2

            lax.fori_loop(0, WB // 2, pair_body, 0)
            return carry

        lax.fori_loop(0, SW, win_body, 0)
        plsc.subcore_barrier()

        pltpu.sync_copy(
            acc.at[pl.ds(sid * RPS, RPS)],
            out_hbm.at[cid, pl.ds(sid * RPS, RPS)],
        )

        @pl.when(sid == NS - 1)
        def _drain_tail():
            pltpu.sync_copy(
                acc.at[pl.ds(NS * RPS, TAIL)],
                out_hbm.at[cid, pl.ds(NS * RPS, TAIL)],
            )

    return k(support, src3, dst3, w3)


def kernel(x, edge_index, edge_weight, W1, b1, a1, W2, b2, a2):
    N, F = x.shape
    E = edge_weight.shape[0]
    NB = -(-E // (NW * B * 32)) * 32
    pad = NW * NB * B - E
    src = jnp.concatenate(
        [edge_index[0].astype(jnp.int32), jnp.zeros((pad,), jnp.int32)]
    )
    dst = jnp.concatenate(
        [edge_index[1].astype(jnp.int32), jnp.zeros((pad,), jnp.int32)]
    )
    w = jnp.concatenate([edge_weight, jnp.zeros((pad,), jnp.float32)])
    src3 = src.reshape(NW, NB * B)
    dst3 = dst.reshape(NW, NB, B)
    w3 = w.reshape(NW, NB * B)

    sup1 = _mm_bias(x, W1, b1)
    parts1 = _sc_spmm(sup1, src3, dst3, w3)
    sup2 = _combine_prelu_mm(parts1[0], parts1[1], a1, W2, b2)
    parts2 = _sc_spmm(sup2, src3, dst3, w3)
    return _combine_prelu(parts2[0], parts2[1], a2)


# speedup vs baseline: 2.6041x; 2.4889x over previous
import functools

import jax
import jax.numpy as jnp
from jax import lax
from jax.experimental import pallas as pl
from jax.experimental.pallas import tpu as pltpu
from jax.experimental.pallas import tpu_sc as plsc

NC = 2
NS = 16
L = 16
NW = NC * NS
B = 80


def _mm_bias(x, W, b):
    M, K = x.shape
    Nf = W.shape[1]
    BM = 2000

    def body(x_ref, W_ref, b_ref, o_ref):
        o_ref[...] = (
            jnp.dot(x_ref[...], W_ref[...], preferred_element_type=jnp.float32)
            + b_ref[...]
        )

    return pl.pallas_call(
        body,
        grid=(M // BM,),
        in_specs=[
            pl.BlockSpec((BM, K), lambda i: (i, 0)),
            pl.BlockSpec((K, Nf), lambda i: (0, 0)),
            pl.BlockSpec((1, Nf), lambda i: (0, 0)),
        ],
        out_specs=pl.BlockSpec((BM, Nf), lambda i: (i, 0)),
        out_shape=jax.ShapeDtypeStruct((M, Nf), jnp.float32),
    )(x, W, b.reshape(1, Nf))


def _combine_prelu_mm(p0, p1, a, W, b):
    M, K = p0.shape
    Nf = W.shape[1]
    BM = 2000

    def body(p0_ref, p1_ref, a_ref, W_ref, b_ref, o_ref):
        h = p0_ref[...] + p1_ref[...]
        h = jnp.where(h >= 0, h, a_ref[...] * h)
        o_ref[...] = (
            jnp.dot(h, W_ref[...], preferred_element_type=jnp.float32) + b_ref[...]
        )

    return pl.pallas_call(
        body,
        grid=(M // BM,),
        in_specs=[
            pl.BlockSpec((BM, K), lambda i: (i, 0)),
            pl.BlockSpec((BM, K), lambda i: (i, 0)),
            pl.BlockSpec((1, K), lambda i: (0, 0)),
            pl.BlockSpec((K, Nf), lambda i: (0, 0)),
            pl.BlockSpec((1, Nf), lambda i: (0, 0)),
        ],
        out_specs=pl.BlockSpec((BM, Nf), lambda i: (i, 0)),
        out_shape=jax.ShapeDtypeStruct((M, Nf), jnp.float32),
    )(p0, p1, a.reshape(1, K), W, b.reshape(1, Nf))


def _combine_prelu(p0, p1, a):
    M, K = p0.shape
    BM = 2000

    def body(p0_ref, p1_ref, a_ref, o_ref):
        h = p0_ref[...] + p1_ref[...]
        o_ref[...] = jnp.where(h >= 0, h, a_ref[...] * h)

    return pl.pallas_call(
        body,
        grid=(M // BM,),
        in_specs=[
            pl.BlockSpec((BM, K), lambda i: (i, 0)),
            pl.BlockSpec((BM, K), lambda i: (i, 0)),
            pl.BlockSpec((1, K), lambda i: (0, 0)),
        ],
        out_specs=pl.BlockSpec((BM, K), lambda i: (i, 0)),
        out_shape=jax.ShapeDtypeStruct((M, K), jnp.float32),
    )(p0, p1, a.reshape(1, K))


def _sc_spmm(support, src3, dst3, w3):
    N, F = support.shape
    NB = dst3.shape[1]
    WB = 32
    SW = NB // WB
    RPS = (N // NS) // 8 * 8
    TAIL = N - NS * RPS
    ZR = 48
    mesh = plsc.VectorSubcoreMesh(core_axis_name="c", subcore_axis_name="s")

    @functools.partial(
        pl.kernel,
        out_type=jax.ShapeDtypeStruct((NC, N, F), jnp.float32),
        mesh=mesh,
        scratch_types=[
            pltpu.VMEM_SHARED((N, F), jnp.float32),
            pltpu.VMEM((NB * B,), jnp.int32),
            pltpu.VMEM((WB, B), jnp.int32),
            pltpu.VMEM((NB * B,), jnp.float32),
            pltpu.VMEM((B, F), jnp.float32),
            pltpu.VMEM((B, F), jnp.float32),
            pltpu.SemaphoreType.DMA,
            pltpu.SemaphoreType.DMA,
        ],
    )
    def k(sup_hbm, src_hbm, dst_hbm, w_hbm, out_hbm,
          acc, src_v, dw0, w_v, rows0, rows1, gs0, gs1):
        cid = lax.axis_index("c")
        sid = lax.axis_index("s")
        wid = cid * NS + sid

        pltpu.sync_copy(src_hbm.at[wid], src_v)
        pltpu.sync_copy(w_hbm.at[wid], w_v)

        zero = jnp.zeros((L,), jnp.float32)

        def zb(r, carry):
            for kk in range(F // L):
                rows0[r, pl.ds(kk * L, L)] = zero
            return carry

        lax.fori_loop(0, ZR, zb, 0)
        for i in range(RPS // ZR):
            pltpu.sync_copy(
                rows0.at[pl.ds(0, ZR)], acc.at[pl.ds(sid * RPS + i * ZR, ZR)]
            )

        @pl.when(sid == NS - 1)
        def _zero_tail():
            pltpu.sync_copy(rows0.at[pl.ds(0, TAIL)], acc.at[pl.ds(NS * RPS, TAIL)])

        plsc.subcore_barrier()

        def gather(j, rows, sem):
            return pltpu.async_copy(sup_hbm.at[src_v.at[pl.ds(j * B, B)]], rows, sem)

        def gwait(rows, sem):
            pltpu.make_async_copy(
                sup_hbm.at[src_v.at[pl.ds(0, B)]], rows, sem
            ).wait()

        def scale(rows, j):
            def group_body(g, c2):
                wchunk = w_v[pl.ds(j * B + g * L, L)]
                for i in range(L):
                    wb = jnp.broadcast_to(wchunk[i], (L,))
                    b = g * L + i
                    for kk in range(F // L):
                        sl = pl.ds(kk * L, L)
                        rows[b, sl] = rows[b, sl] * wb
                return c2

            lax.fori_loop(0, B // L, group_body, 0)

        def win_body(w, carry):
            woff = pl.multiple_of(w * WB, WB)
            pltpu.sync_copy(dst_hbm.at[wid, pl.ds(woff, WB)], dw0)
            base = w * WB

            def pair_body(p, carry2):
                j0 = base + 2 * p
                d0 = gather(j0, rows0, gs0)
                d1 = gather(j0 + 1, rows1, gs1)
                d0.wait()
                scale(rows0, j0)
                pltpu.sync_copy(rows0, acc.at[dw0.at[2 * p]], add=True)

                d1.wait()
                scale(rows1, j0 + 1)
                pltpu.sync_copy(rows1, acc.at[dw0.at[2 * p + 1]], add=True)
                return carry2

            lax.fori_loop(0, WB // 2, pair_body, 0)
            return carry

        lax.fori_loop(0, SW, win_body, 0)
        plsc.subcore_barrier()

        pltpu.sync_copy(
            acc.at[pl.ds(sid * RPS, RPS)],
            out_hbm.at[cid, pl.ds(sid * RPS, RPS)],
        )

        @pl.when(sid == NS - 1)
        def _drain_tail():
            pltpu.sync_copy(
                acc.at[pl.ds(NS * RPS, TAIL)],
                out_hbm.at[cid, pl.ds(NS * RPS, TAIL)],
            )

    return k(support, src3, dst3, w3)


def kernel(x, edge_index, edge_weight, W1, b1, a1, W2, b2, a2):
    N, F = x.shape
    E = edge_weight.shape[0]
    NB = -(-E // (NW * B * 32)) * 32
    pad = NW * NB * B - E
    spread = jnp.arange(pad, dtype=jnp.int32) % N
    src = jnp.concatenate([edge_index[0].astype(jnp.int32), spread])
    dst = jnp.concatenate([edge_index[1].astype(jnp.int32), spread])
    w = jnp.concatenate([edge_weight, jnp.zeros((pad,), jnp.float32)])
    src3 = src.reshape(NW, NB * B)
    dst3 = dst.reshape(NW, NB, B)
    w3 = w.reshape(NW, NB * B)

    sup1 = _mm_bias(x, W1, b1)
    parts1 = _sc_spmm(sup1, src3, dst3, w3)
    sup2 = _combine_prelu_mm(parts1[0], parts1[1], a1, W2, b2)
    parts2 = _sc_spmm(sup2, src3, dst3, w3)
    return _combine_prelu(parts2[0], parts2[1], a2)


# speedup vs baseline: 3.6130x; 1.3874x over previous
import functools

import jax
import jax.numpy as jnp
from jax import lax
from jax.experimental import pallas as pl
from jax.experimental.pallas import tpu as pltpu
from jax.experimental.pallas import tpu_sc as plsc

NC = 2
NS = 16
L = 16
NW = NC * NS
B = 80


def _mm_bias(x, W, b):
    M, K = x.shape
    Nf = W.shape[1]
    BM = 2000

    def body(x_ref, W_ref, b_ref, o_ref):
        o_ref[...] = (
            jnp.dot(x_ref[...], W_ref[...], preferred_element_type=jnp.float32)
            + b_ref[...]
        )

    return pl.pallas_call(
        body,
        grid=(M // BM,),
        in_specs=[
            pl.BlockSpec((BM, K), lambda i: (i, 0)),
            pl.BlockSpec((K, Nf), lambda i: (0, 0)),
            pl.BlockSpec((1, Nf), lambda i: (0, 0)),
        ],
        out_specs=pl.BlockSpec((BM, Nf), lambda i: (i, 0)),
        out_shape=jax.ShapeDtypeStruct((M, Nf), jnp.float32),
    )(x, W, b.reshape(1, Nf))


def _combine_prelu_mm(p0, p1, a, W, b):
    M, K = p0.shape
    Nf = W.shape[1]
    BM = 2000

    def body(p0_ref, p1_ref, a_ref, W_ref, b_ref, o_ref):
        h = p0_ref[...] + p1_ref[...]
        h = jnp.where(h >= 0, h, a_ref[...] * h)
        o_ref[...] = (
            jnp.dot(h, W_ref[...], preferred_element_type=jnp.float32) + b_ref[...]
        )

    return pl.pallas_call(
        body,
        grid=(M // BM,),
        in_specs=[
            pl.BlockSpec((BM, K), lambda i: (i, 0)),
            pl.BlockSpec((BM, K), lambda i: (i, 0)),
            pl.BlockSpec((1, K), lambda i: (0, 0)),
            pl.BlockSpec((K, Nf), lambda i: (0, 0)),
            pl.BlockSpec((1, Nf), lambda i: (0, 0)),
        ],
        out_specs=pl.BlockSpec((BM, Nf), lambda i: (i, 0)),
        out_shape=jax.ShapeDtypeStruct((M, Nf), jnp.float32),
    )(p0, p1, a.reshape(1, K), W, b.reshape(1, Nf))


def _combine_prelu(p0, p1, a):
    M, K = p0.shape
    BM = 2000

    def body(p0_ref, p1_ref, a_ref, o_ref):
        h = p0_ref[...] + p1_ref[...]
        o_ref[...] = jnp.where(h >= 0, h, a_ref[...] * h)

    return pl.pallas_call(
        body,
        grid=(M // BM,),
        in_specs=[
            pl.BlockSpec((BM, K), lambda i: (i, 0)),
            pl.BlockSpec((BM, K), lambda i: (i, 0)),
            pl.BlockSpec((1, K), lambda i: (0, 0)),
        ],
        out_specs=pl.BlockSpec((BM, K), lambda i: (i, 0)),
        out_shape=jax.ShapeDtypeStruct((M, K), jnp.float32),
    )(p0, p1, a.reshape(1, K))


def _sc_spmm(support, src3, dst3, w3):
    N, F = support.shape
    NB = dst3.shape[1]
    WB = 32
    SW = NB // WB
    RPS = (N // NS) // 8 * 8
    TAIL = N - NS * RPS
    ZR = 48
    mesh = plsc.VectorSubcoreMesh(core_axis_name="c", subcore_axis_name="s")

    @functools.partial(
        pl.kernel,
        out_type=jax.ShapeDtypeStruct((NC, N, F), jnp.float32),
        mesh=mesh,
        scratch_types=[
            pltpu.VMEM_SHARED((N, F), jnp.float32),
            pltpu.VMEM((NB * B,), jnp.int32),
            pltpu.VMEM((WB, B), jnp.int32),
            pltpu.VMEM((NB * B,), jnp.float32),
            pltpu.VMEM((B, F), jnp.float32),
            pltpu.VMEM((B, F), jnp.float32),
            pltpu.SemaphoreType.DMA,
            pltpu.SemaphoreType.DMA,
        ],
    )
    def k(sup_hbm, src_hbm, dst_hbm, w_hbm, out_hbm,
          acc, src_v, dw0, w_v, rows0, rows1, gs0, gs1):
        cid = lax.axis_index("c")
        sid = lax.axis_index("s")
        wid = cid * NS + sid

        pltpu.sync_copy(src_hbm.at[wid], src_v)
        pltpu.sync_copy(w_hbm.at[wid], w_v)

        zero = jnp.zeros((L,), jnp.float32)

        def zb(r, carry):
            for kk in range(F // L):
                rows0[r, pl.ds(kk * L, L)] = zero
            return carry

        lax.fori_loop(0, ZR, zb, 0)
        for i in range(RPS // ZR):
            pltpu.sync_copy(
                rows0.at[pl.ds(0, ZR)], acc.at[pl.ds(sid * RPS + i * ZR, ZR)]
            )

        @pl.when(sid == NS - 1)
        def _zero_tail():
            pltpu.sync_copy(rows0.at[pl.ds(0, TAIL)], acc.at[pl.ds(NS * RPS, TAIL)])

        plsc.subcore_barrier()

        def gather(j, rows, sem):
            return pltpu.async_copy(sup_hbm.at[src_v.at[pl.ds(j * B, B)]], rows, sem)

        def gwait(rows, sem):
            pltpu.make_async_copy(
                sup_hbm.at[src_v.at[pl.ds(0, B)]], rows, sem
            ).wait()

        def scale(rows, j):
            def group_body(g, c2):
                wchunk = w_v[pl.ds(j * B + g * L, L)]
                for i in range(L):
                    wb = jnp.broadcast_to(wchunk[i], (L,))
                    b = g * L + i
                    for kk in range(F // L):
                        sl = pl.ds(kk * L, L)
                        rows[b, sl] = rows[b, sl] * wb
                return c2

            lax.fori_loop(0, B // L, group_body, 0)

        gather(0, rows0, gs0)

        def win_body(w, carry):
            woff = pl.multiple_of(w * WB, WB)
            pltpu.sync_copy(dst_hbm.at[wid, pl.ds(woff, WB)], dw0)
            base = w * WB

            def pair_body(p, carry2):
                j0 = base + 2 * p
                gather(j0 + 1, rows1, gs1)
                gwait(rows0, gs0)
                scale(rows0, j0)
                pltpu.sync_copy(rows0, acc.at[dw0.at[2 * p]], add=True)

                @pl.when(j0 + 2 < NB)
                def _():
                    gather(j0 + 2, rows0, gs0)

                gwait(rows1, gs1)
                scale(rows1, j0 + 1)
                pltpu.sync_copy(rows1, acc.at[dw0.at[2 * p + 1]], add=True)
                return carry2

            lax.fori_loop(0, WB // 2, pair_body, 0)
            return carry

        lax.fori_loop(0, SW, win_body, 0)
        plsc.subcore_barrier()

        pltpu.sync_copy(
            acc.at[pl.ds(sid * RPS, RPS)],
            out_hbm.at[cid, pl.ds(sid * RPS, RPS)],
        )

        @pl.when(sid == NS - 1)
        def _drain_tail():
            pltpu.sync_copy(
                acc.at[pl.ds(NS * RPS, TAIL)],
                out_hbm.at[cid, pl.ds(NS * RPS, TAIL)],
            )

    return k(support, src3, dst3, w3)


def kernel(x, edge_index, edge_weight, W1, b1, a1, W2, b2, a2):
    N, F = x.shape
    E = edge_weight.shape[0]
    NB = -(-E // (NW * B * 32)) * 32
    pad = NW * NB * B - E
    spread = jnp.arange(pad, dtype=jnp.int32) % N
    src = jnp.concatenate([edge_index[0].astype(jnp.int32), spread])
    dst = jnp.concatenate([edge_index[1].astype(jnp.int32), spread])
    w = jnp.concatenate([edge_weight, jnp.zeros((pad,), jnp.float32)])
    src3 = src.reshape(NW, NB * B)
    dst3 = dst.reshape(NW, NB, B)
    w3 = w.reshape(NW, NB * B)

    sup1 = _mm_bias(x, W1, b1)
    parts1 = _sc_spmm(sup1, src3, dst3, w3)
    sup2 = _combine_prelu_mm(parts1[0], parts1[1], a1, W2, b2)
    parts2 = _sc_spmm(sup2, src3, dst3, w3)
    return _combine_prelu(parts2[0], parts2[1], a2)
